# Initial kernel scaffold; baseline (speedup 1.0000x reference)
import jax
import jax.numpy as jnp
from jax.experimental import pallas as pl


def kernel(x, edge_attr, params, edge_index, batch):
    raise NotImplementedError("write your pallas kernel here")


# speedup vs baseline: 3.8371x; 3.8371x over previous
import jax
import jax.numpy as jnp
from jax import lax
from jax.experimental import pallas as pl
from jax.experimental.pallas import tpu as pltpu

BSZ = 64
NPG = 64
EPG = 1024
IN_FEAT = 128
HID = 128
EF = 16
HEADS = 8
POS = NPG * NPG
F32 = jnp.float32


def _dot(a, b):
    return jnp.dot(a, b, preferred_element_type=F32)


def _graph_kernel(x_ref, ea_ref, srcc_ref, dstc_ref, dstr_ref, keyc_ref,
                  keyr_ref, atomW_ref, atomb_ref, bondW_ref, bondb_ref,
                  gW_ref, gb_ref, W1a_ref, W1e_ref, b1_ref, W2_ref, b2_ref,
                  out_ref):
    h = _dot(x_ref[...], atomW_ref[...]) + atomb_ref[...]
    ea = _dot(ea_ref[...], bondW_ref[...]) + bondb_ref[...]

    srcc = srcc_ref[0]
    dstc = dstc_ref[0]
    dstr = dstr_ref[0]
    col_n = lax.broadcasted_iota(jnp.int32, (EPG, NPG), 1)
    src_oh = jnp.where(srcc == col_n, 1.0, 0.0)
    dst_oh = jnp.where(dstc == col_n, 1.0, 0.0)
    row_n = lax.broadcasted_iota(jnp.int32, (NPG, EPG), 0)
    dst_ohT = jnp.where(row_n == dstr, 1.0, 0.0)

    for l in range(3):
        t = jax.nn.relu(_dot(ea, gW_ref[4 * l]) + gb_ref[4 * l])
        e = _dot(t, gW_ref[4 * l + 1]) + gb_ref[4 * l + 1]
        msg = jax.nn.relu(_dot(src_oh, h) + e)
        aggr = _dot(dst_ohT, msg)
        h2 = h + aggr
        h2 = jax.nn.relu(_dot(h2, gW_ref[4 * l + 2]) + gb_ref[4 * l + 2])
        h2 = _dot(h2, gW_ref[4 * l + 3]) + gb_ref[4 * l + 3]
        if l < 2:
            h2 = jax.nn.relu(h2)
        h = h2

    W1a = W1a_ref[...]
    b1 = b1_ref[...]
    W2 = W2_ref[...]
    b2 = b2_ref[...]
    sel = jnp.where(
        lax.broadcasted_iota(jnp.int32, (HID, HEADS), 0) % HEADS
        == lax.broadcasted_iota(jnp.int32, (HID, HEADS), 1), 1.0, 0.0)

    hn = jnp.broadcast_to(h[:, None, :], (NPG, NPG, HID)).reshape(POS, HID)
    hm = jnp.broadcast_to(h[None, :, :], (NPG, NPG, HID)).reshape(POS, HID)
    attn8 = _dot(hn * hm, sel)
    zd = _dot(attn8, W1a) + b1
    outd = _dot(jax.nn.relu(zd), W2) + b2

    hs = _dot(src_oh, h)
    hd = _dot(dst_oh, h)
    attn8e = _dot(hs * hd, sel)
    ze = _dot(attn8e, W1a) + b1 + _dot(ea, W1e_ref[...])
    ve = _dot(jax.nn.relu(ze), W2) + b2

    keyc = keyc_ref[0]
    keyr = keyr_ref[0]
    e_row = lax.broadcasted_iota(jnp.int32, (EPG, EPG), 0)
    e_col = lax.broadcasted_iota(jnp.int32, (EPG, EPG), 1)
    later_dup = (keyc == keyr) & (e_row > e_col)
    dupcnt = jnp.sum(jnp.where(later_dup, 1.0, 0.0), axis=0, keepdims=True)
    keptr = jnp.where(dupcnt == 0.0, 1.0, 0.0)

    ve2 = jnp.concatenate([ve, jnp.ones((EPG, 1), F32)], axis=1)
    pieces = []
    for c in range(4):
        jrow = lax.broadcasted_iota(jnp.int32, (EPG, EPG), 0) + c * EPG
        McT = jnp.where(jrow == keyr, 1.0, 0.0) * keptr
        pieces.append(_dot(McT, ve2))
    scat = jnp.concatenate(pieces, axis=0)
    out_ref[...] = jnp.where(scat[:, 1:2] > 0.5, scat[:, 0:1], outd)


def kernel(x, edge_attr, params, edge_index, batch):
    src = edge_index[0]
    dst = edge_index[1]
    src_l = jnp.remainder(src, NPG).astype(jnp.int32)
    dst_l = jnp.remainder(dst, NPG).astype(jnp.int32)
    key = src_l * NPG + dst_l
    srcc = src_l.reshape(BSZ, EPG, 1)
    dstc = dst_l.reshape(BSZ, EPG, 1)
    dstr = dst_l.reshape(BSZ, 1, EPG)
    keyc = key.reshape(BSZ, EPG, 1)
    keyr = key.reshape(BSZ, 1, EPG)

    p = params
    gWs = jnp.stack([p['gnn'][l][k] for l in range(3)
                     for k in ('be_W1', 'be_W2', 'nn_W1', 'nn_W2')])
    gbs = jnp.stack([p['gnn'][l][k] for l in range(3)
                     for k in ('be_b1', 'be_b2', 'nn_b1', 'nn_b2')])
    gbs = gbs.reshape(12, 1, HID)
    fd = HID // HEADS
    W1a = p['mlp_W1'][:HEADS] * (1.0 / (fd ** 0.5))
    W1e = p['mlp_W1'][HEADS:]
    b1 = p['mlp_b1'].reshape(1, HID)
    W2 = p['mlp_W2']
    b2 = p['mlp_b2'].reshape(1, 1)
    atomb = p['atom_b'].reshape(1, HID)
    bondb = p['bond_b'].reshape(1, HID)

    full2 = lambda g: (0, 0)
    full3 = lambda g: (0, 0, 0)
    out = pl.pallas_call(
        _graph_kernel,
        grid=(BSZ,),
        in_specs=[
            pl.BlockSpec((NPG, IN_FEAT), lambda g: (g, 0)),
            pl.BlockSpec((EPG, EF), lambda g: (g, 0)),
            pl.BlockSpec((1, EPG, 1), lambda g: (g, 0, 0)),
            pl.BlockSpec((1, EPG, 1), lambda g: (g, 0, 0)),
            pl.BlockSpec((1, 1, EPG), lambda g: (g, 0, 0)),
            pl.BlockSpec((1, EPG, 1), lambda g: (g, 0, 0)),
            pl.BlockSpec((1, 1, EPG), lambda g: (g, 0, 0)),
            pl.BlockSpec((IN_FEAT, HID), full2),
            pl.BlockSpec((1, HID), full2),
            pl.BlockSpec((EF, HID), full2),
            pl.BlockSpec((1, HID), full2),
            pl.BlockSpec((12, HID, HID), full3),
            pl.BlockSpec((12, 1, HID), full3),
            pl.BlockSpec((HEADS, HID), full2),
            pl.BlockSpec((HID, HID), full2),
            pl.BlockSpec((1, HID), full2),
            pl.BlockSpec((HID, 1), full2),
            pl.BlockSpec((1, 1), full2),
        ],
        out_specs=pl.BlockSpec((POS, 1), lambda g: (g, 0)),
        out_shape=jax.ShapeDtypeStruct((BSZ * POS, 1), F32),
    )(x, edge_attr, srcc, dstc, dstr, keyc, keyr,
      p['atom_W'], atomb, p['bond_W'], bondb, gWs, gbs,
      W1a, W1e, b1, W2, b2)

    emb = out.reshape(BSZ, NPG, NPG, 1)
    mask = jnp.ones((BSZ, NPG, NPG), F32)
    return emb, mask
